# baseline (speedup 1.0000x reference)
import functools

import jax
import jax.numpy as jnp
from jax import lax
from jax.experimental import pallas as pl
from jax.experimental.pallas import tpu as pltpu
from jax.experimental.pallas import tpu_sc as plsc

_NC = 2
_NS = 16
_NW = _NC * _NS
_L = 16


@functools.lru_cache(maxsize=None)
def _build(B, NUMF, CATF, V, D):
    NTOK = 1 + NUMF + CATF
    NPAIR = NTOK // 2
    NNUM = (1 + NUMF) // 2
    NCAT = CATF // 2
    D2 = 2 * D
    BPW = B // _NW
    SUB = 16
    NSUB = BPW // SUB
    ND = D // _L
    NB = BPW // _L

    mesh = plsc.VectorSubcoreMesh(core_axis_name="c", subcore_axis_name="s")

    @functools.partial(
        pl.kernel,
        out_type=jax.ShapeDtypeStruct((B * NPAIR, D2), jnp.float32),
        mesh=mesh,
        scratch_types=[
            pltpu.VMEM((NUMF, D), jnp.float32),
            pltpu.VMEM((NUMF, D), jnp.float32),
            pltpu.VMEM((D,), jnp.float32),
            pltpu.VMEM((NUMF, BPW), jnp.float32),
            pltpu.VMEM((CATF, BPW), jnp.int32),
            pltpu.VMEM((CATF, BPW), jnp.int32),
            pltpu.VMEM((4, BPW, D2), jnp.float32),
            pltpu.VMEM((BPW, D2), jnp.float32),
            pltpu.VMEM((NNUM * SUB, D2), jnp.float32),
            pltpu.VMEM((1, BPW), jnp.int32),
            pltpu.VMEM((1, NNUM * SUB), jnp.int32),
            pltpu.SemaphoreType.DMA,
            pltpu.SemaphoreType.DMA,
        ],
    )
    def sc_kernel(tab_hbm, catT_hbm, numT_hbm, w_hbm, bias_hbm, cls_hbm,
                  out_hbm, wv, bv, clsv, numv, idxm, idxg, ring, catbuf,
                  numbuf, didx, nidx, gsem, wsem):
        wid = lax.axis_index("s") * _NC + lax.axis_index("c")
        base = wid * BPW
        iota = lax.broadcasted_iota(jnp.int32, (_L,), 0)

        pltpu.sync_copy(catT_hbm.at[:, pl.ds(base, BPW)], idxm)
        for c in range(CATF):
            off = c * V
            for p in range(NB):
                sl = pl.ds(p * _L, _L)
                idxg[c, sl] = idxm[c, sl] + off

        def start_gather(c, slot):
            return pltpu.async_copy(
                tab_hbm.at[idxg.at[c]], ring.at[slot], gsem)

        g0 = start_gather(0, 0)
        g1 = start_gather(1, 1)

        pltpu.sync_copy(w_hbm, wv)
        pltpu.sync_copy(bias_hbm, bv)
        pltpu.sync_copy(cls_hbm, clsv)
        pltpu.sync_copy(numT_hbm.at[:, pl.ds(base, BPW)], numv)

        for j in range(SUB):
            for dd in range(ND):
                sl = pl.ds(dd * _L, _L)
                numbuf[j, sl] = clsv[sl]

        def num_body(s, carry):
            for i in range(NUMF):
                t = 1 + i
                roff = (t // 2) * SUB
                hoff = (t % 2) * D
                vec = numv[i, pl.ds(s * SUB, SUB)]
                for j in range(SUB):
                    x = vec[j]
                    for dd in range(ND):
                        numbuf[roff + j, pl.ds(hoff + dd * _L, _L)] = (
                            wv[i, pl.ds(dd * _L, _L)] * x
                            + bv[i, pl.ds(dd * _L, _L)])
            for p in range(NNUM):
                b0 = (base + s * SUB) * NPAIR + p
                nidx[0, pl.ds(p * SUB, SUB)] = iota * NPAIR + b0
            pltpu.async_copy(numbuf, out_hbm.at[nidx.at[0]], wsem).wait()
            return carry

        lax.fori_loop(0, NSUB, num_body, 0)

        def cat_body(q, carry):
            c0 = 2 * q
            slot = lax.rem(q, 2) * 2

            @pl.when(q == 0)
            def _w0():
                g0.wait()
                g1.wait()

            @pl.when(q > 0)
            def _w1():
                for _ in range(2):
                    pltpu.make_async_copy(
                        tab_hbm.at[pl.ds(0, BPW)], ring.at[0], gsem).wait()

            @pl.when(q + 1 < NCAT)
            def _prefetch():
                nslot = lax.rem(q + 1, 2) * 2
                pltpu.async_copy(
                    tab_hbm.at[idxg.at[2 * q + 2]], ring.at[nslot], gsem)
                pltpu.async_copy(
                    tab_hbm.at[idxg.at[2 * q + 3]], ring.at[nslot + 1], gsem)

            def blk_body(jj, c2):
                for dd in range(ND):
                    catbuf[jj, pl.ds(dd * _L, _L)] = ring[
                        slot, jj, pl.ds(dd * _L, _L)]
                    catbuf[jj, pl.ds(D + dd * _L, _L)] = ring[
                        slot + 1, jj, pl.ds(dd * _L, _L)]
                return c2

            lax.fori_loop(0, BPW, blk_body, 0)

            for p in range(NB):
                sl = pl.ds(p * _L, _L)
                didx[0, sl] = iota * NPAIR + ((base + p * _L) * NPAIR + NNUM + q)
            pltpu.async_copy(catbuf, out_hbm.at[didx.at[0]], wsem).wait()
            return carry

        lax.fori_loop(0, NCAT, cat_body, 0)

    return sc_kernel


def kernel(numerical, categorical, W_num, b_num, tables, cls_token):
    B, NUMF = numerical.shape
    CATF = categorical.shape[1]
    V, D = tables.shape[1], tables.shape[2]
    NTOK = 1 + NUMF + CATF
    tab2d = tables.reshape(CATF * V, D)
    tab_pair = jnp.broadcast_to(tab2d[:, None, :], (CATF * V, 2, D)).reshape(
        CATF * V, 2 * D)
    cat_t = categorical.T.astype(jnp.int32)
    num_t = numerical.T
    cls_vec = cls_token.reshape(D)
    fn = _build(B, NUMF, CATF, V, D)
    out_pair = fn(tab_pair, cat_t, num_t, W_num, b_num, cls_vec)
    return out_pair.reshape(B, NTOK, D)


# speedup vs baseline: 16.2955x; 16.2955x over previous
import functools

import jax
import jax.numpy as jnp
from jax import lax
from jax.experimental import pallas as pl
from jax.experimental.pallas import tpu as pltpu
from jax.experimental.pallas import tpu_sc as plsc

_NC = 2
_NS = 16
_NW = _NC * _NS
_L = 16


@functools.lru_cache(maxsize=None)
def _build(B, NUMF, CATF, V, D):
    NTOK = 1 + NUMF + CATF
    NPAIR = NTOK // 2
    NNUM = (1 + NUMF) // 2
    NCAT = CATF // 2
    D2 = 2 * D
    BPW = B // _NW
    SUB = 16
    NSUB = BPW // SUB
    ND = D // _L
    NB = BPW // _L

    mesh = plsc.VectorSubcoreMesh(core_axis_name="c", subcore_axis_name="s")

    @functools.partial(
        pl.kernel,
        out_type=jax.ShapeDtypeStruct((B * NPAIR, D2), jnp.float32),
        mesh=mesh,
        scratch_types=[
            pltpu.VMEM((NUMF, D), jnp.float32),
            pltpu.VMEM((NUMF, D), jnp.float32),
            pltpu.VMEM((D,), jnp.float32),
            pltpu.VMEM((NUMF, BPW), jnp.float32),
            pltpu.VMEM((CATF, BPW), jnp.int32),
            pltpu.VMEM((CATF, BPW), jnp.int32),
            pltpu.VMEM((4, BPW, D2), jnp.float32),
            pltpu.VMEM((BPW, D2), jnp.float32),
            pltpu.VMEM((NNUM * SUB, D2), jnp.float32),
            pltpu.VMEM((1, BPW), jnp.int32),
            pltpu.VMEM((1, NNUM * SUB), jnp.int32),
            pltpu.SemaphoreType.DMA,
            pltpu.SemaphoreType.DMA,
        ],
    )
    def sc_kernel(tab_hbm, catT_hbm, numT_hbm, w_hbm, bias_hbm, cls_hbm,
                  out_hbm, wv, bv, clsv, numv, idxm, idxg, ring, catbuf,
                  numbuf, didx, nidx, gsem, wsem):
        wid = lax.axis_index("s") * _NC + lax.axis_index("c")
        base = wid * BPW
        iota = lax.broadcasted_iota(jnp.int32, (_L,), 0)

        pltpu.sync_copy(catT_hbm.at[:, pl.ds(base, BPW)], idxm)
        for c in range(CATF):
            off = c * V
            for p in range(NB):
                sl = pl.ds(p * _L, _L)
                idxg[c, sl] = idxm[c, sl] + off

        def start_gather(c, slot):
            return pltpu.async_copy(
                tab_hbm.at[idxg.at[c]], ring.at[slot], gsem)

        g0 = start_gather(0, 0)
        g1 = start_gather(1, 1)

        pltpu.sync_copy(w_hbm, wv)
        pltpu.sync_copy(bias_hbm, bv)
        pltpu.sync_copy(cls_hbm, clsv)
        pltpu.sync_copy(numT_hbm.at[:, pl.ds(base, BPW)], numv)

        for j in range(SUB):
            for dd in range(ND):
                sl = pl.ds(dd * _L, _L)
                numbuf[j, sl] = clsv[sl]

        def num_body(s, carry):
            for i in range(NUMF):
                t = 1 + i
                roff = (t // 2) * SUB
                hoff = (t % 2) * D
                vec = numv[i, pl.ds(s * SUB, SUB)]
                for j in range(SUB):
                    x = vec[j]
                    for dd in range(ND):
                        numbuf[roff + j, pl.ds(hoff + dd * _L, _L)] = (
                            wv[i, pl.ds(dd * _L, _L)] * x
                            + bv[i, pl.ds(dd * _L, _L)])
            for p in range(NNUM):
                b0 = (base + s * SUB) * NPAIR + p
                nidx[0, pl.ds(p * SUB, SUB)] = iota * NPAIR + b0
            pltpu.async_copy(numbuf, out_hbm.at[nidx.at[0]], wsem).wait()
            return carry

        lax.fori_loop(0, NSUB, num_body, 0)

        def cat_body(q, carry):
            c0 = 2 * q
            slot = lax.rem(q, 2) * 2

            @pl.when(q == 0)
            def _w0():
                g0.wait()
                g1.wait()

            @pl.when(q > 0)
            def _w1():
                for _ in range(2):
                    pltpu.make_async_copy(
                        tab_hbm.at[pl.ds(0, BPW)], ring.at[0], gsem).wait()

            @pl.when(q + 1 < NCAT)
            def _prefetch():
                nslot = lax.rem(q + 1, 2) * 2
                pltpu.async_copy(
                    tab_hbm.at[idxg.at[2 * q + 2]], ring.at[nslot], gsem)
                pltpu.async_copy(
                    tab_hbm.at[idxg.at[2 * q + 3]], ring.at[nslot + 1], gsem)

            def blk_body(jj, c2):
                for dd in range(ND):
                    catbuf[jj, pl.ds(dd * _L, _L)] = ring[
                        slot, jj, pl.ds(dd * _L, _L)]
                    catbuf[jj, pl.ds(D + dd * _L, _L)] = ring[
                        slot + 1, jj, pl.ds(dd * _L, _L)]
                return c2

            lax.fori_loop(0, BPW, blk_body, 0)

            for p in range(NB):
                sl = pl.ds(p * _L, _L)
                didx[0, sl] = iota * NPAIR + ((base + p * _L) * NPAIR + NNUM + q)
            pltpu.async_copy(catbuf, out_hbm.at[didx.at[0]], wsem).wait()
            return carry

        lax.fori_loop(0, NCAT, cat_body, 0)

    return sc_kernel


def kernel(numerical, categorical, W_num, b_num, tables, cls_token):
    B, NUMF = numerical.shape
    CATF = categorical.shape[1]
    V, D = tables.shape[1], tables.shape[2]
    NTOK = 1 + NUMF + CATF
    tab_pair = jnp.pad(tables.reshape(CATF * V, D), ((0, 0), (0, D)))
    cat_t = categorical.T.astype(jnp.int32)
    num_t = numerical.T
    cls_vec = cls_token.reshape(D)
    fn = _build(B, NUMF, CATF, V, D)
    out_pair = fn(tab_pair, cat_t, num_t, W_num, b_num, cls_vec)
    return out_pair.reshape(B, NTOK, D)


# speedup vs baseline: 29.9867x; 1.8402x over previous
import functools

import jax
import jax.numpy as jnp
from jax import lax
from jax.experimental import pallas as pl
from jax.experimental.pallas import tpu as pltpu
from jax.experimental.pallas import tpu_sc as plsc

_NC = 2
_NS = 16
_NW = _NC * _NS
_L = 16


@functools.lru_cache(maxsize=None)
def _build(B, NUMF, CATF, V, D):
    NTOK = 1 + NUMF + CATF
    NPAIR = NTOK // 2
    NNUM = (1 + NUMF) // 2
    NCAT = CATF // 2
    D2 = 2 * D
    BPW = B // _NW
    SUB = 16
    NSUB = BPW // SUB
    ND = D // _L
    NB = BPW // _L

    mesh = plsc.VectorSubcoreMesh(core_axis_name="c", subcore_axis_name="s")

    @functools.partial(
        pl.kernel,
        out_type=jax.ShapeDtypeStruct((B * NPAIR, D2), jnp.float32),
        mesh=mesh,
        scratch_types=[
            pltpu.VMEM((NUMF, D), jnp.float32),
            pltpu.VMEM((NUMF, D), jnp.float32),
            pltpu.VMEM((D,), jnp.float32),
            pltpu.VMEM((NUMF, BPW), jnp.float32),
            pltpu.VMEM((CATF, BPW), jnp.int32),
            pltpu.VMEM((CATF, BPW), jnp.int32),
            pltpu.VMEM((2 * _L, 8, D), jnp.float32),
            pltpu.VMEM((BPW, D2), jnp.float32),
            pltpu.VMEM((NNUM * SUB, D2), jnp.float32),
            pltpu.VMEM((1, BPW), jnp.int32),
            pltpu.VMEM((1, NNUM * SUB), jnp.int32),
            pltpu.SemaphoreType.DMA,
            pltpu.SemaphoreType.DMA,
        ],
    )
    def sc_kernel(tab_hbm, catT_hbm, numT_hbm, w_hbm, bias_hbm, cls_hbm,
                  out_hbm, wv, bv, clsv, numv, g8m, r8m, ring, catbuf,
                  numbuf, didx, nidx, gsem, wsem):
        wid = lax.axis_index("s") * _NC + lax.axis_index("c")
        base = wid * BPW
        iota = lax.broadcasted_iota(jnp.int32, (_L,), 0)

        pltpu.sync_copy(catT_hbm.at[:, pl.ds(base, BPW)], g8m)
        for c in range(CATF):
            off = c * V
            for p in range(NB):
                sl = pl.ds(p * _L, _L)
                idx = g8m[c, sl] + off
                g8m[c, sl] = lax.bitwise_and(idx, ~7)
                r8m[c, sl] = lax.bitwise_and(idx, 7)

        pltpu.sync_copy(w_hbm, wv)
        pltpu.sync_copy(bias_hbm, bv)
        pltpu.sync_copy(cls_hbm, clsv)
        pltpu.sync_copy(numT_hbm.at[:, pl.ds(base, BPW)], numv)

        for j in range(SUB):
            for dd in range(ND):
                sl = pl.ds(dd * _L, _L)
                numbuf[j, sl] = clsv[sl]

        def num_body(s, carry):
            for i in range(NUMF):
                t = 1 + i
                roff = (t // 2) * SUB
                hoff = (t % 2) * D
                vec = numv[i, pl.ds(s * SUB, SUB)]
                for j in range(SUB):
                    x = vec[j]
                    for dd in range(ND):
                        numbuf[roff + j, pl.ds(hoff + dd * _L, _L)] = (
                            wv[i, pl.ds(dd * _L, _L)] * x
                            + bv[i, pl.ds(dd * _L, _L)])
            for p in range(NNUM):
                b0 = (base + s * SUB) * NPAIR + p
                nidx[0, pl.ds(p * SUB, SUB)] = iota * NPAIR + b0
            pltpu.async_copy(numbuf, out_hbm.at[nidx.at[0]], wsem).wait()
            return carry

        lax.fori_loop(0, NSUB, num_body, 0)

        def fetch_block(c, p):
            g8v = g8m[c, pl.ds(p * _L, _L)]
            sbase = lax.rem(p, 2) * _L
            for j in range(_L):
                g8 = pl.multiple_of(g8v[j], 8)
                pltpu.async_copy(
                    tab_hbm.at[pl.ds(g8, 8), :], ring.at[sbase + j], gsem)

        def extract_block(c, p, half):
            r8v = r8m[c, pl.ds(p * _L, _L)]
            sbase = lax.rem(p, 2) * _L
            for j in range(_L):
                pltpu.make_async_copy(
                    tab_hbm.at[pl.ds(0, 8), :], ring.at[0], gsem).wait()
                r = r8v[j]
                for dd in range(ND):
                    catbuf[p * _L + j, pl.ds(half * D + dd * _L, _L)] = ring[
                        sbase + j, r, pl.ds(dd * _L, _L)]

        def cat_body(q, carry):
            for half in range(2):
                c = 2 * q + half

                def blk_body(p, c2):
                    @pl.when(p < NB)
                    def _issue():
                        fetch_block(c, p)

                    @pl.when(p > 0)
                    def _drain():
                        extract_block(c, p - 1, half)

                    return c2

                lax.fori_loop(0, NB + 1, blk_body, 0)

            for p in range(NB):
                sl = pl.ds(p * _L, _L)
                didx[0, sl] = iota * NPAIR + ((base + p * _L) * NPAIR + NNUM + q)
            pltpu.async_copy(catbuf, out_hbm.at[didx.at[0]], wsem).wait()
            return carry

        lax.fori_loop(0, NCAT, cat_body, 0)

    return sc_kernel


def kernel(numerical, categorical, W_num, b_num, tables, cls_token):
    B, NUMF = numerical.shape
    CATF = categorical.shape[1]
    V, D = tables.shape[1], tables.shape[2]
    NTOK = 1 + NUMF + CATF
    tab_flat = tables.reshape(CATF * V, D)
    cat_t = categorical.T.astype(jnp.int32)
    num_t = numerical.T
    cls_vec = cls_token.reshape(D)
    fn = _build(B, NUMF, CATF, V, D)
    out_pair = fn(tab_flat, cat_t, num_t, W_num, b_num, cls_vec)
    return out_pair.reshape(B, NTOK, D)


# speedup vs baseline: 32.8849x; 1.0967x over previous
import functools

import jax
import jax.numpy as jnp
from jax import lax
from jax.experimental import pallas as pl
from jax.experimental.pallas import tpu as pltpu
from jax.experimental.pallas import tpu_sc as plsc

_NC = 2
_NS = 16
_NW = _NC * _NS
_L = 16


@functools.lru_cache(maxsize=None)
def _build(B, NUMF, CATF, V, D):
    NTOK = 1 + NUMF + CATF
    BPW = B // _NW
    SUB = 16
    NSUB = BPW // SUB
    ND = D // _L

    mesh = plsc.VectorSubcoreMesh(core_axis_name="c", subcore_axis_name="s")

    @functools.partial(
        pl.kernel,
        out_type=jax.ShapeDtypeStruct((B, NTOK // 2, 2 * D), jnp.float32),
        mesh=mesh,
        scratch_types=[
            pltpu.VMEM((NUMF, D), jnp.float32),
            pltpu.VMEM((NUMF, D), jnp.float32),
            pltpu.VMEM((D,), jnp.float32),
            pltpu.VMEM((NUMF, BPW), jnp.float32),
            pltpu.VMEM((CATF, BPW), jnp.int32),
            pltpu.VMEM((CATF, BPW), jnp.int32),
            pltpu.VMEM((2 * _L, 8, D), jnp.float32),
            pltpu.VMEM((1, SUB, NTOK // 2, 2 * D), jnp.float32),
            pltpu.SemaphoreType.DMA,
            pltpu.SemaphoreType.DMA,
        ],
    )
    def sc_kernel(tab_hbm, catT_hbm, numT_hbm, w_hbm, bias_hbm, cls_hbm,
                  out_hbm, wv, bv, clsv, numv, g8m, r8m, ring, stg,
                  gsem, wsem):
        wid = lax.axis_index("s") * _NC + lax.axis_index("c")
        base = wid * BPW

        pltpu.sync_copy(catT_hbm.at[:, pl.ds(base, BPW)], g8m)
        for c in range(CATF):
            off = c * V
            for p in range(BPW // _L):
                sl = pl.ds(p * _L, _L)
                idx = g8m[c, sl] + off
                g8m[c, sl] = lax.bitwise_and(idx, ~7)
                r8m[c, sl] = lax.bitwise_and(idx, 7)

        pltpu.sync_copy(w_hbm, wv)
        pltpu.sync_copy(bias_hbm, bv)
        pltpu.sync_copy(cls_hbm, clsv)
        pltpu.sync_copy(numT_hbm.at[:, pl.ds(base, BPW)], numv)

        def chunk_body(s, carry):
            dbuf = 0

            for j in range(SUB):
                for dd in range(ND):
                    sl = pl.ds(dd * _L, _L)
                    stg[dbuf, j, 0, sl] = clsv[sl]
            for i in range(NUMF):
                t = 1 + i
                pr = t // 2
                hoff = (t % 2) * D
                vec = numv[i, pl.ds(s * SUB, SUB)]
                for j in range(SUB):
                    x = vec[j]
                    for dd in range(ND):
                        sl = pl.ds(dd * _L, _L)
                        stg[dbuf, j, pr, pl.ds(hoff + dd * _L, _L)] = (
                            wv[i, sl] * x + bv[i, sl])

            def blk_body(c, c2):
                @pl.when(c < CATF)
                def _issue():
                    g8v = g8m[c, pl.ds(s * SUB, SUB)]
                    sb = lax.rem(c, 2) * _L
                    for j in range(_L):
                        g8 = pl.multiple_of(g8v[j], 8)
                        pltpu.async_copy(
                            tab_hbm.at[pl.ds(g8, 8), :], ring.at[sb + j], gsem)

                @pl.when(c > 0)
                def _extract():
                    cp = c - 1
                    r8v = r8m[cp, pl.ds(s * SUB, SUB)]
                    sb = lax.rem(cp, 2) * _L
                    pr = (1 + NUMF + cp) // 2
                    hoff = lax.rem(cp, 2) * D
                    for j in range(_L):
                        pltpu.make_async_copy(
                            tab_hbm.at[pl.ds(0, 8), :], ring.at[0], gsem).wait()
                        r = r8v[j]
                        for dd in range(ND):
                            sl = pl.ds(dd * _L, _L)
                            stg[dbuf, j, pr, pl.ds(hoff + dd * _L, _L)] = ring[
                                sb + j, r, sl]

                return c2

            lax.fori_loop(0, CATF + 1, blk_body, 0)

            pltpu.async_copy(
                stg.at[dbuf], out_hbm.at[pl.ds(base + s * SUB, SUB)], wsem
            ).wait()
            return carry

        lax.fori_loop(0, NSUB, chunk_body, 0)

    return sc_kernel


def kernel(numerical, categorical, W_num, b_num, tables, cls_token):
    B, NUMF = numerical.shape
    CATF = categorical.shape[1]
    V, D = tables.shape[1], tables.shape[2]
    NTOK = 1 + NUMF + CATF
    tab_flat = tables.reshape(CATF * V, D)
    cat_t = categorical.T.astype(jnp.int32)
    num_t = numerical.T
    cls_vec = cls_token.reshape(D)
    fn = _build(B, NUMF, CATF, V, D)
    out_pair = fn(tab_flat, cat_t, num_t, W_num, b_num, cls_vec)
    return out_pair.reshape(B, NTOK, D)
